# baseline (speedup 1.0000x reference)
import jax
import jax.numpy as jnp
from jax.experimental import pallas as pl

D_MODEL = 1024
SEQ = 4
PERIOD = 8
GROUPS_PER_BLOCK = 64


def _body(x_ref, e2_ref, e4_ref, e8_ref, o_ref):
    t = (
        jnp.tile(e2_ref[...], (4, 1))
        + jnp.tile(e4_ref[...], (2, 1))
        + e8_ref[...]
    )
    add = jnp.tile(t, (1, SEQ))
    o_ref[...] = x_ref[...] + add[None, :, :]


def kernel(x, emb2, emb4, emb8):
    L = x.shape[0]
    g = L // PERIOD
    xv = x.reshape(g, PERIOD, SEQ * D_MODEL)
    grid = (g // GROUPS_PER_BLOCK,)
    out = pl.pallas_call(
        _body,
        grid=grid,
        in_specs=[
            pl.BlockSpec((GROUPS_PER_BLOCK, PERIOD, SEQ * D_MODEL), lambda i: (i, 0, 0)),
            pl.BlockSpec((2, D_MODEL), lambda i: (0, 0)),
            pl.BlockSpec((4, D_MODEL), lambda i: (0, 0)),
            pl.BlockSpec((8, D_MODEL), lambda i: (0, 0)),
        ],
        out_specs=pl.BlockSpec((GROUPS_PER_BLOCK, PERIOD, SEQ * D_MODEL), lambda i: (i, 0, 0)),
        out_shape=jax.ShapeDtypeStruct((g, PERIOD, SEQ * D_MODEL), x.dtype),
    )(xv, emb2, emb4, emb8)
    return out.reshape(L, SEQ, D_MODEL)
